# Initial kernel scaffold; baseline (speedup 1.0000x reference)
import jax
import jax.numpy as jnp
from jax.experimental import pallas as pl


def kernel(x, edge_index, h_msg, Wi, bi, We, be, Wm1, bm1, Wm2, bm2, Wn1, bn1, Wn2, bn2, Wu, bu, Wd, bd, Wb, bb):
    raise NotImplementedError("write your pallas kernel here")


# speedup vs baseline: 2.3747x; 2.3747x over previous
import functools

import jax
import jax.numpy as jnp
from jax import lax
from jax.experimental import pallas as pl
from jax.experimental.pallas import tpu as pltpu
from jax.experimental.pallas import tpu_sc as plsc

N = 100000
E = 1600000
H = 32
HALF = N // 2


def _lrelu(v):
    return jnp.where(v > 0, v, 0.01 * v)


def _node_embed(x, Wi, bi):
    BN = 4000

    def body(x_ref, wi_ref, bi_ref, out_ref):
        out_ref[...] = (
            jnp.dot(x_ref[...], wi_ref[...], preferred_element_type=jnp.float32)
            + bi_ref[...]
        )

    return pl.pallas_call(
        body,
        grid=(N // BN,),
        in_specs=[
            pl.BlockSpec((BN, 3), lambda i: (i, 0)),
            pl.BlockSpec((3, H), lambda i: (0, 0)),
            pl.BlockSpec((1, H), lambda i: (0, 0)),
        ],
        out_specs=pl.BlockSpec((BN, H), lambda i: (i, 0)),
        out_shape=jax.ShapeDtypeStruct((N, H), jnp.float32),
    )(x, Wi, bi.reshape(1, H))


def _sc_gather(table, eidx):
    CH = 1024
    NCHUNK = (2 * E) // CH
    NW = 32
    PER_W = -(-NCHUNK // NW)
    mesh = plsc.VectorSubcoreMesh(core_axis_name="c", subcore_axis_name="s")

    @functools.partial(
        pl.kernel,
        mesh=mesh,
        compiler_params=pltpu.CompilerParams(use_tc_tiling_on_sc=False),
        out_type=jax.ShapeDtypeStruct((2 * E, H), jnp.float32),
        scratch_types=[
            pltpu.VMEM((CH,), jnp.int32),
            pltpu.VMEM((CH, H), jnp.float32),
            pltpu.SemaphoreType.DMA,
        ],
    )
    def k(tab, idx_hbm, out, idxv, rows, sem):
        w = lax.axis_index("s") * 2 + lax.axis_index("c")

        def body(i, carry):
            cid = w + NW * i

            @pl.when(cid < NCHUNK)
            def _():
                base = pl.multiple_of(cid * CH, CH)
                pltpu.sync_copy(idx_hbm.at[pl.ds(base, CH)], idxv)
                cps = [
                    pltpu.async_copy(
                        tab.at[idxv.at[pl.ds(s * 128, 128)]],
                        rows.at[pl.ds(s * 128, 128), :],
                        sem,
                    )
                    for s in range(CH // 128)
                ]
                for cp in cps:
                    cp.wait()
                pltpu.sync_copy(rows, out.at[pl.ds(base, CH), :])

            return carry

        lax.fori_loop(0, PER_W, body, 0)

    return k(table, eidx)


def _sc_scatter(h_new, dst):
    CH = 512
    NCHUNK = E // CH
    PER_T = -(-NCHUNK // 16)
    R = 50048
    ZCH = R // 128
    mesh = plsc.VectorSubcoreMesh(core_axis_name="c", subcore_axis_name="s")

    @functools.partial(
        pl.kernel,
        mesh=mesh,
        compiler_params=pltpu.CompilerParams(use_tc_tiling_on_sc=False),
        out_type=jax.ShapeDtypeStruct((N, H), jnp.float32),
        scratch_types=[
            pltpu.VMEM((CH,), jnp.int32),
            pltpu.VMEM((CH // 128, 128), jnp.int32),
            pltpu.VMEM((CH, H), jnp.float32),
            pltpu.VMEM_SHARED((R, H), jnp.float32),
            pltpu.SemaphoreType.DMA,
        ],
    )
    def k(hnew, dref, out, dbuf, ibuf, rbuf, table, sem):
        c = lax.axis_index("c")
        s = lax.axis_index("s")
        nbase = c * HALF
        zeros16 = jnp.zeros((16,), jnp.float32)

        def zrow(r, carry):
            rbuf[r, pl.ds(0, 16)] = zeros16
            rbuf[r, pl.ds(16, 16)] = zeros16
            return carry

        lax.fori_loop(0, 128, zrow, 0)

        def ztab(i, carry):
            z = s + 16 * i

            @pl.when(z < ZCH)
            def _():
                off = pl.multiple_of(z * 128, 128)
                pltpu.sync_copy(rbuf.at[pl.ds(0, 128), :], table.at[pl.ds(off, 128), :])

            return carry

        lax.fori_loop(0, -(-ZCH // 16), ztab, 0)
        plsc.subcore_barrier()

        def chunk(i, carry):
            cid = s + 16 * i

            @pl.when(cid < NCHUNK)
            def _():
                base = pl.multiple_of(cid * CH, CH)
                pltpu.sync_copy(dref.at[pl.ds(base, CH)], dbuf)
                pltpu.sync_copy(hnew.at[pl.ds(base, CH), :], rbuf)
                for j in range(CH // 16):
                    v = dbuf[pl.ds(j * 16, 16)]
                    loc = v - nbase
                    ok = (loc >= 0) & (loc < HALF)
                    loc = jnp.where(ok, loc, HALF)
                    ibuf[j // 8, pl.ds((j % 8) * 16, 16)] = loc
                for t in range(CH // 128):
                    pltpu.sync_copy(
                        rbuf.at[pl.ds(t * 128, 128), :],
                        table.at[ibuf.at[t]],
                        add=True,
                    )

            return carry

        lax.fori_loop(0, PER_T, chunk, 0)
        plsc.subcore_barrier()

        rows_per_t = HALF // 16
        roff = s * rows_per_t
        pltpu.sync_copy(
            table.at[pl.ds(roff, rows_per_t), :],
            out.at[pl.ds(nbase + roff, rows_per_t), :],
        )

    return k(h_new, dst)


def _edge_compute(hsrc, hdst, hmsg, We, be, Wm1, bm1, Wm2, bm2,
                  Wn1, bn1, Wn2, bn2, Wd, bd):
    EB = 6400

    def body(hs_ref, hd_ref, hm_ref, we_ref, be_ref, wm1_ref, bm1_ref,
             wm2_ref, bm2_ref, wn1_ref, bn1_ref, wn2_ref, bn2_ref,
             wd_ref, bd_ref, hnew_ref, y_ref):
        f32 = jnp.float32
        hs = hs_ref[...]
        hd = hd_ref[...]
        hm = hm_ref[...]
        wm1 = wm1_ref[...]
        wa = wm1[0:H]
        wb = wm1[H:2 * H]
        wc = wm1[2 * H:3 * H]
        wem = jnp.dot(we_ref[...], wc, preferred_element_type=f32)
        bpre = jnp.dot(be_ref[...], wc, preferred_element_type=f32) + bm1_ref[...]
        pre = (
            jnp.dot(hd, wa, preferred_element_type=f32)
            + jnp.dot(hs, wb, preferred_element_type=f32)
            + jnp.dot(hm, wem, preferred_element_type=f32)
            + bpre
        )
        m = _lrelu(pre)
        m = _lrelu(jnp.dot(m, wm2_ref[...], preferred_element_type=f32) + bm2_ref[...])
        nmid = _lrelu(jnp.dot(hs, wn1_ref[...], preferred_element_type=f32) + bn1_ref[...])
        n = _lrelu(jnp.dot(nmid, wn2_ref[...], preferred_element_type=f32) + bn2_ref[...])
        hn = m + n
        hnew_ref[...] = hn
        z = jnp.dot(hn, wd_ref[...], preferred_element_type=f32) + bd_ref[...]
        z = z - jnp.max(z, axis=-1, keepdims=True)
        ez = jnp.exp(z)
        y_ref[...] = ez / jnp.sum(ez, axis=-1, keepdims=True)

    wspec = pl.BlockSpec((H, H), lambda i: (0, 0))
    bspec = pl.BlockSpec((1, H), lambda i: (0, 0))
    return pl.pallas_call(
        body,
        grid=(E // EB,),
        in_specs=[
            pl.BlockSpec((EB, H), lambda i: (i, 0)),
            pl.BlockSpec((EB, H), lambda i: (i, 0)),
            pl.BlockSpec((EB, H), lambda i: (i, 0)),
            wspec, bspec,
            pl.BlockSpec((3 * H, H), lambda i: (0, 0)), bspec,
            wspec, bspec,
            wspec, bspec,
            wspec, bspec,
            pl.BlockSpec((H, 2), lambda i: (0, 0)),
            pl.BlockSpec((1, 2), lambda i: (0, 0)),
        ],
        out_specs=[
            pl.BlockSpec((EB, H), lambda i: (i, 0)),
            pl.BlockSpec((EB, 2), lambda i: (i, 0)),
        ],
        out_shape=[
            jax.ShapeDtypeStruct((E, H), jnp.float32),
            jax.ShapeDtypeStruct((E, 2), jnp.float32),
        ],
    )(hsrc, hdst, hmsg, We, be.reshape(1, H), Wm1, bm1.reshape(1, H),
      Wm2, bm2.reshape(1, H), Wn1, bn1.reshape(1, H), Wn2, bn2.reshape(1, H),
      Wd, bd.reshape(1, 2))


def _beliefs(h2, a2, Wu, bu, Wb, bb):
    BN = 2000

    def body(h_ref, a_ref, wu_ref, bu_ref, wb_ref, bb_ref, out_ref):
        f32 = jnp.float32
        hn = h_ref[:, 1, :]
        ag = a_ref[:, 1, :]
        wu = wu_ref[...]
        z = (
            jnp.dot(hn, wu[0:H], preferred_element_type=f32)
            + jnp.dot(ag, wu[H:2 * H], preferred_element_type=f32)
            + bu_ref[...]
        )
        z = _lrelu(z)
        t = jnp.dot(z, wb_ref[...], preferred_element_type=f32) + bb_ref[...]
        t = t - jnp.max(t, axis=-1, keepdims=True)
        et = jnp.exp(t)
        out_ref[...] = et / jnp.sum(et, axis=-1, keepdims=True)

    return pl.pallas_call(
        body,
        grid=(HALF // BN,),
        in_specs=[
            pl.BlockSpec((BN, 2, H), lambda i: (i, 0, 0)),
            pl.BlockSpec((BN, 2, H), lambda i: (i, 0, 0)),
            pl.BlockSpec((2 * H, H), lambda i: (0, 0)),
            pl.BlockSpec((1, H), lambda i: (0, 0)),
            pl.BlockSpec((H, 2), lambda i: (0, 0)),
            pl.BlockSpec((1, 2), lambda i: (0, 0)),
        ],
        out_specs=pl.BlockSpec((BN, 2), lambda i: (i, 0)),
        out_shape=jax.ShapeDtypeStruct((HALF, 2), jnp.float32),
    )(h2, a2, Wu, bu.reshape(1, H), Wb, bb.reshape(1, 2))


def kernel(x, edge_index, h_msg, Wi, bi, We, be, Wm1, bm1, Wm2, bm2,
           Wn1, bn1, Wn2, bn2, Wu, bu, Wd, bd, Wb, bb):
    h_node = _node_embed(x, Wi, bi)

    eidx = edge_index.reshape(2 * E)
    gath = _sc_gather(h_node, eidx)
    hsrc = gath[:E]
    hdst = gath[E:]

    h_new, y_msg = _edge_compute(hsrc, hdst, h_msg, We, be, Wm1, bm1, Wm2, bm2,
                                 Wn1, bn1, Wn2, bn2, Wd, bd)

    aggr = _sc_scatter(h_new, edge_index[1])

    y_beliefs = _beliefs(h_node.reshape(HALF, 2, H), aggr.reshape(HALF, 2, H),
                         Wu, bu, Wb, bb)
    return (h_new, y_msg, y_beliefs)
